# baseline (speedup 1.0000x reference)
import functools

import jax
import jax.numpy as jnp
from jax import lax
from jax.experimental import pallas as pl
from jax.experimental.pallas import tpu as pltpu
from jax.experimental.pallas import tpu_sc as plsc

B = 16384
NUM_NUMERIC = 13
N_CAT = 26
CARD = 100000
D = 32
N_TOK = 1 + NUM_NUMERIC + N_CAT
FH = N_CAT // 2

NC = 2
NS = 16
NW = NC * NS
B_PER_W = B // NW
NB = 64
N_NCHUNK = B_PER_W // NB


def _splat_at(ref2d, i, j):
    isplat = jnp.full((16,), i, dtype=jnp.int32)
    jsplat = jnp.full((16,), j, dtype=jnp.int32)
    return plsc.load_gather(ref2d, [isplat, jsplat])


def _numcls(base, num_v, numbuf_v, w_v, b_v, cls_v, out_hbm):
    def nchunk(c, carry):
        b0 = base + c * NB

        def cls_d(d, carry2):
            v = _splat_at(cls_v, 0, d)
            for g in range(NB // 16):
                numbuf_v[0, d, pl.ds(g * 16, 16)] = v
            return carry2
        lax.fori_loop(0, D, cls_d, 0)

        def num_i(i, carry2):
            def num_d(d, carry3):
                w_id = _splat_at(w_v, i, d)
                b_id = _splat_at(b_v, i, d)
                for g in range(NB // 16):
                    nv = num_v[i, pl.ds(c * NB + g * 16, 16)]
                    numbuf_v[i + 1, d, pl.ds(g * 16, 16)] = nv * w_id + b_id
                return carry3
            lax.fori_loop(0, D, num_d, 0)
            return carry2
        lax.fori_loop(0, NUM_NUMERIC, num_i, 0)

        pltpu.sync_copy(
            numbuf_v, out_hbm.at[pl.ds(0, 1 + NUM_NUMERIC), :, pl.ds(b0, NB)])
        return carry
    lax.fori_loop(0, N_NCHUNK, nchunk, 0)


def _gather_half(tok0, cat_slab, tbl_hbm, out_hbm, idx_v, bufs, base, gsem,
                 osem):
    def fire_field(f, outbuf_v):
        def fire_d(d, carry):
            pltpu.async_copy(tbl_hbm.at[f, d].at[idx_v.at[f]],
                             outbuf_v.at[d], gsem)
            return carry
        lax.fori_loop(0, D, fire_d, 0)

    def drain_field(f, outbuf_v):
        def wait_d(d, carry):
            pltpu.make_async_copy(tbl_hbm.at[f, d].at[idx_v.at[f]],
                                  outbuf_v.at[d], gsem).wait()
            return carry
        lax.fori_loop(0, D, wait_d, 0)

    def out_slab(f):
        return out_hbm.at[tok0 + f, :, pl.ds(base, B_PER_W)]

    for f in range(3):
        fire_field(f, bufs[f])
    cat_slab()
    for f in range(FH):
        if f >= 1:
            pltpu.make_async_copy(bufs[(f - 1) % 4], out_slab(f - 1),
                                  osem).wait()
        if f + 3 < FH:
            fire_field(f + 3, bufs[(f + 3) % 4])
        drain_field(f, bufs[f % 4])
        pltpu.async_copy(bufs[f % 4], out_slab(f), osem)

    pltpu.make_async_copy(bufs[(FH - 1) % 4], out_slab(FH - 1), osem).wait()


def _kernel_a(cat_t_hbm, tbl_hbm, num_t_hbm, w_hbm, bias_hbm, cls_hbm,
              out_hbm,
              idx_v, num_v, numbuf_v, outbuf_a, outbuf_b, outbuf_c,
              outbuf_d, w_v, b_v, cls_v,
              gsem, osem):
    wid = lax.axis_index("s") * NC + lax.axis_index("c")
    base = wid * B_PER_W

    pltpu.sync_copy(w_hbm, w_v)
    pltpu.sync_copy(bias_hbm, b_v)
    pltpu.sync_copy(cls_hbm, cls_v)
    pltpu.sync_copy(cat_t_hbm.at[:, pl.ds(base, B_PER_W)], idx_v)
    pltpu.sync_copy(num_t_hbm.at[:, pl.ds(base, B_PER_W)], num_v)

    bufs = [outbuf_a, outbuf_b, outbuf_c, outbuf_d]
    numcls = functools.partial(_numcls, base, num_v, numbuf_v, w_v, b_v,
                               cls_v, out_hbm)
    _gather_half(1 + NUM_NUMERIC, numcls, tbl_hbm, out_hbm, idx_v, bufs,
                 base, gsem, osem)


def _kernel_b(cat_t_hbm, tbl_hbm, out_hbm,
              idx_v, outbuf_a, outbuf_b, outbuf_c, outbuf_d,
              gsem, osem):
    wid = lax.axis_index("s") * NC + lax.axis_index("c")
    base = wid * B_PER_W

    pltpu.sync_copy(cat_t_hbm.at[:, pl.ds(base, B_PER_W)], idx_v)

    bufs = [outbuf_a, outbuf_b, outbuf_c, outbuf_d]
    _gather_half(0, lambda: None, tbl_hbm, out_hbm, idx_v, bufs, base,
                 gsem, osem)


_OUTBUFS = [pltpu.VMEM((D, B_PER_W), jnp.float32)] * 4


@jax.jit
def kernel(numeric, categorical, num_weight, num_bias, cat_tables, cls_token):
    cat_t = categorical.T
    num_t = numeric.T
    tbl_sw = jnp.swapaxes(cat_tables, 1, 2)

    mesh = plsc.VectorSubcoreMesh(core_axis_name="c", subcore_axis_name="s")
    cparams = pltpu.CompilerParams(use_tc_tiling_on_sc=False,
                                   needs_layout_passes=False)

    run_a = pl.kernel(
        _kernel_a,
        out_type=jax.ShapeDtypeStruct((1 + NUM_NUMERIC + FH, D, B),
                                      jnp.float32),
        mesh=mesh,
        compiler_params=cparams,
        scratch_types=[
            pltpu.VMEM((FH, B_PER_W), jnp.int32),
            pltpu.VMEM((NUM_NUMERIC, B_PER_W), jnp.float32),
            pltpu.VMEM((1 + NUM_NUMERIC, D, NB), jnp.float32),
            *_OUTBUFS,
            pltpu.VMEM((NUM_NUMERIC, D), jnp.float32),
            pltpu.VMEM((NUM_NUMERIC, D), jnp.float32),
            pltpu.VMEM((1, D), jnp.float32),
            pltpu.SemaphoreType.DMA,
            pltpu.SemaphoreType.DMA,
        ],
    )
    run_b = pl.kernel(
        _kernel_b,
        out_type=jax.ShapeDtypeStruct((FH, D, B), jnp.float32),
        mesh=mesh,
        compiler_params=cparams,
        scratch_types=[
            pltpu.VMEM((FH, B_PER_W), jnp.int32),
            *_OUTBUFS,
            pltpu.SemaphoreType.DMA,
            pltpu.SemaphoreType.DMA,
        ],
    )
    out_a = run_a(cat_t[:FH], tbl_sw[:FH], num_t, num_weight, num_bias,
                  cls_token.reshape(1, D))
    out_b = run_b(cat_t[FH:], tbl_sw[FH:])
    out_p = jnp.concatenate([out_a, out_b], axis=0)
    return out_p.transpose(2, 0, 1)
